# baseline (speedup 1.0000x reference)
import jax
import jax.numpy as jnp
from jax import lax
from jax.experimental import pallas as pl
from jax.experimental.pallas import tpu as pltpu
from jax.experimental.pallas import tpu_sc as plsc

STU_NUM = 100000
PP_DIM = 128
BATCH = 16384
X_DIM = 128
OUT_DIM = PP_DIM + X_DIM

NC = 2
NS = 16
NW = NC * NS
B_PER_W = BATCH // NW

C = 64
NCHUNK = B_PER_W // C
NBUF = 6


def _gather_concat(x_hbm, idx_hbm, table_hbm, out_hbm, idx_v, out_v,
                   gsem, xsem, wsem):
    wid = lax.axis_index("s") * NC + lax.axis_index("c")
    base = wid * B_PER_W
    pltpu.sync_copy(idx_hbm.at[pl.ds(base, B_PER_W)], idx_v)

    def fire(k, b):
        pltpu.async_copy(
            table_hbm.at[idx_v.at[pl.ds(k * C, C)]],
            out_v.at[b, :, pl.ds(0, PP_DIM)], gsem.at[b])
        pltpu.async_copy(
            x_hbm.at[pl.ds(base + k * C, C)],
            out_v.at[b, :, pl.ds(PP_DIM, X_DIM)], xsem.at[b])

    def wait_in(k, b):
        pltpu.make_async_copy(
            table_hbm.at[idx_v.at[pl.ds(k * C, C)]],
            out_v.at[b, :, pl.ds(0, PP_DIM)], gsem.at[b]).wait()
        pltpu.make_async_copy(
            x_hbm.at[pl.ds(base + k * C, C)],
            out_v.at[b, :, pl.ds(PP_DIM, X_DIM)], xsem.at[b]).wait()

    def wait_out(k, b):
        pltpu.make_async_copy(
            out_v.at[b], out_hbm.at[pl.ds(base + k * C, C)], wsem.at[b]).wait()

    for k in range(min(NBUF, NCHUNK)):
        fire(k, k % NBUF)
    for k in range(NCHUNK):
        b = k % NBUF
        wait_in(k, b)
        pltpu.async_copy(out_v.at[b], out_hbm.at[pl.ds(base + k * C, C)],
                         wsem.at[b])
        nk = k + NBUF
        if nk < NCHUNK:
            wait_out(k, b)
            fire(nk, b)
    for k in range(max(0, NCHUNK - NBUF), NCHUNK):
        wait_out(k, k % NBUF)


@jax.jit
def _run(x, stu_id, ability_emb):
    mesh = plsc.VectorSubcoreMesh(core_axis_name="c", subcore_axis_name="s")
    return pl.kernel(
        _gather_concat,
        out_type=jax.ShapeDtypeStruct((BATCH, OUT_DIM), jnp.float32),
        mesh=mesh,
        scratch_types=[
            pltpu.VMEM((B_PER_W,), jnp.int32),
            pltpu.VMEM((NBUF, C, OUT_DIM), jnp.float32),
            pltpu.SemaphoreType.DMA((NBUF,)),
            pltpu.SemaphoreType.DMA((NBUF,)),
            pltpu.SemaphoreType.DMA((NBUF,)),
        ],
    )(x, stu_id, ability_emb)


def kernel(x, stu_id, ability_emb):
    return _run(x, stu_id.astype(jnp.int32), ability_emb)


# speedup vs baseline: 1.0122x; 1.0122x over previous
import jax
import jax.numpy as jnp
from jax import lax
from jax.experimental import pallas as pl
from jax.experimental.pallas import tpu as pltpu
from jax.experimental.pallas import tpu_sc as plsc

STU_NUM = 100000
PP_DIM = 128
BATCH = 16384
X_DIM = 128
OUT_DIM = PP_DIM + X_DIM

NC = 2
NS = 16
NW = NC * NS
B_PER_W = BATCH // NW

C = 128
NCHUNK = B_PER_W // C
NBUF = 3


def _gather_concat(x_hbm, idx_hbm, table_hbm, out_hbm, idx_v, out_v,
                   gsem, xsem, wsem):
    wid = lax.axis_index("s") * NC + lax.axis_index("c")
    base = wid * B_PER_W
    pltpu.sync_copy(idx_hbm.at[pl.ds(base, B_PER_W)], idx_v)

    def fire(k, b):
        pltpu.async_copy(
            table_hbm.at[idx_v.at[pl.ds(k * C, C)]],
            out_v.at[b, :, pl.ds(0, PP_DIM)], gsem.at[b])
        pltpu.async_copy(
            x_hbm.at[pl.ds(base + k * C, C)],
            out_v.at[b, :, pl.ds(PP_DIM, X_DIM)], xsem.at[b])

    def wait_in(k, b):
        pltpu.make_async_copy(
            table_hbm.at[idx_v.at[pl.ds(k * C, C)]],
            out_v.at[b, :, pl.ds(0, PP_DIM)], gsem.at[b]).wait()
        pltpu.make_async_copy(
            x_hbm.at[pl.ds(base + k * C, C)],
            out_v.at[b, :, pl.ds(PP_DIM, X_DIM)], xsem.at[b]).wait()

    def wait_out(k, b):
        pltpu.make_async_copy(
            out_v.at[b], out_hbm.at[pl.ds(base + k * C, C)], wsem.at[b]).wait()

    for k in range(min(NBUF, NCHUNK)):
        fire(k, k % NBUF)
    for k in range(NCHUNK):
        b = k % NBUF
        wait_in(k, b)
        pltpu.async_copy(out_v.at[b], out_hbm.at[pl.ds(base + k * C, C)],
                         wsem.at[b])
        nk = k + NBUF
        if nk < NCHUNK:
            wait_out(k, b)
            fire(nk, b)
    for k in range(max(0, NCHUNK - NBUF), NCHUNK):
        wait_out(k, k % NBUF)


@jax.jit
def _run(x, stu_id, ability_emb):
    mesh = plsc.VectorSubcoreMesh(core_axis_name="c", subcore_axis_name="s")
    return pl.kernel(
        _gather_concat,
        out_type=jax.ShapeDtypeStruct((BATCH, OUT_DIM), jnp.float32),
        mesh=mesh,
        scratch_types=[
            pltpu.VMEM((B_PER_W,), jnp.int32),
            pltpu.VMEM((NBUF, C, OUT_DIM), jnp.float32),
            pltpu.SemaphoreType.DMA((NBUF,)),
            pltpu.SemaphoreType.DMA((NBUF,)),
            pltpu.SemaphoreType.DMA((NBUF,)),
        ],
    )(x, stu_id, ability_emb)


def kernel(x, stu_id, ability_emb):
    return _run(x, stu_id.astype(jnp.int32), ability_emb)


# speedup vs baseline: 1.0169x; 1.0046x over previous
import jax
import jax.numpy as jnp
from jax import lax
from jax.experimental import pallas as pl
from jax.experimental.pallas import tpu as pltpu
from jax.experimental.pallas import tpu_sc as plsc

STU_NUM = 100000
PP_DIM = 128
BATCH = 16384
X_DIM = 128
OUT_DIM = PP_DIM + X_DIM

NC = 2
NS = 16
NW = NC * NS
B_PER_W = BATCH // NW

C = 128
NCHUNK = B_PER_W // C
NBUF = 3


def _gather_concat(x_hbm, idx_hbm, table_hbm, out_hbm,
                   idx0, idx1, idx2, idx3, rows_v, x_v,
                   isem, gsem, xsem, rsem, wsem):
    idx_bufs = [idx0, idx1, idx2, idx3]
    wid = lax.axis_index("s") * NC + lax.axis_index("c")
    base = wid * B_PER_W
    for k in range(NCHUNK):
        pltpu.async_copy(idx_hbm.at[pl.ds(base + k * C, C)], idx_bufs[k],
                         isem)
    for k in range(NCHUNK):
        pltpu.make_async_copy(idx_hbm.at[pl.ds(base + k * C, C)],
                              idx_bufs[k], isem).wait()

    def fire(k, b):
        pltpu.async_copy(table_hbm.at[idx_bufs[k]], rows_v.at[b], gsem.at[b])
        pltpu.async_copy(x_hbm.at[pl.ds(base + k * C, C)], x_v.at[b],
                         xsem.at[b])

    def wait_in(k, b):
        pltpu.make_async_copy(table_hbm.at[idx_bufs[k]], rows_v.at[b],
                              gsem.at[b]).wait()
        pltpu.make_async_copy(x_hbm.at[pl.ds(base + k * C, C)], x_v.at[b],
                              xsem.at[b]).wait()

    def wait_out(k, b):
        pltpu.make_async_copy(
            rows_v.at[b],
            out_hbm.at[pl.ds(base + k * C, C), pl.ds(0, PP_DIM)],
            rsem.at[b]).wait()
        pltpu.make_async_copy(
            x_v.at[b],
            out_hbm.at[pl.ds(base + k * C, C), pl.ds(PP_DIM, X_DIM)],
            wsem.at[b]).wait()

    for k in range(min(NBUF, NCHUNK)):
        fire(k, k % NBUF)
    for k in range(NCHUNK):
        b = k % NBUF
        wait_in(k, b)
        pltpu.async_copy(
            rows_v.at[b],
            out_hbm.at[pl.ds(base + k * C, C), pl.ds(0, PP_DIM)], rsem.at[b])
        pltpu.async_copy(
            x_v.at[b],
            out_hbm.at[pl.ds(base + k * C, C), pl.ds(PP_DIM, X_DIM)],
            wsem.at[b])
        nk = k + NBUF
        if nk < NCHUNK:
            wait_out(k, b)
            fire(nk, b)
    for k in range(max(0, NCHUNK - NBUF), NCHUNK):
        wait_out(k, k % NBUF)


@jax.jit
def _run(x, stu_id, ability_emb):
    mesh = plsc.VectorSubcoreMesh(core_axis_name="c", subcore_axis_name="s")
    return pl.kernel(
        _gather_concat,
        out_type=jax.ShapeDtypeStruct((BATCH, OUT_DIM), jnp.float32),
        mesh=mesh,
        scratch_types=[
            pltpu.VMEM((C,), jnp.int32),
            pltpu.VMEM((C,), jnp.int32),
            pltpu.VMEM((C,), jnp.int32),
            pltpu.VMEM((C,), jnp.int32),
            pltpu.VMEM((NBUF, C, PP_DIM), jnp.float32),
            pltpu.VMEM((NBUF, C, X_DIM), jnp.float32),
            pltpu.SemaphoreType.DMA,
            pltpu.SemaphoreType.DMA((NBUF,)),
            pltpu.SemaphoreType.DMA((NBUF,)),
            pltpu.SemaphoreType.DMA((NBUF,)),
            pltpu.SemaphoreType.DMA((NBUF,)),
        ],
    )(x, stu_id, ability_emb)


def kernel(x, stu_id, ability_emb):
    return _run(x, stu_id.astype(jnp.int32), ability_emb)
